# baseline (speedup 1.0000x reference)
import functools

import jax
import jax.numpy as jnp
from jax import lax
from jax.experimental import pallas as pl
from jax.experimental.pallas import tpu as pltpu
from jax.experimental.pallas import tpu_sc as plsc

H = 30
W = 30
C = 64
D = 16
K = 3
B = 1024
N = H * W

BB = 8
NC = 2
NS = 16
NW = NC * NS
PER = B // NW
L = 16
NFULL = N // L


def _tc_proj_body(x_ref, wqT_ref, bq_ref, wvT_ref, bv_ref, attn_ref, val_ref):
    wqT = wqT_ref[...]
    wvT = wvT_ref[...]
    bq = bq_ref[...]
    bv = bv_ref[...]
    for b in range(BB):
        x = x_ref[b]
        logits = jnp.dot(wqT, x, preferred_element_type=jnp.float32) + bq
        m = jnp.max(logits, axis=0, keepdims=True)
        e = jnp.exp(logits - m)
        s = jnp.sum(e, axis=0, keepdims=True)
        attn = e / s
        vals = jnp.dot(wvT, x, preferred_element_type=jnp.float32) + bv
        attn_ref[b, pl.ds(0, K), :] = attn
        val_ref[b] = vals


def _tc_project(state3, wqT, bq2, wvT, bv2):
    return pl.pallas_call(
        _tc_proj_body,
        grid=(B // BB,),
        in_specs=[
            pl.BlockSpec((BB, C, N), lambda i: (i, 0, 0)),
            pl.BlockSpec((K, C), lambda i: (0, 0)),
            pl.BlockSpec((K, 1), lambda i: (0, 0)),
            pl.BlockSpec((D, C), lambda i: (0, 0)),
            pl.BlockSpec((D, 1), lambda i: (0, 0)),
        ],
        out_specs=[
            pl.BlockSpec((BB, K + 1, N), lambda i: (i, 0, 0)),
            pl.BlockSpec((BB, D, N), lambda i: (i, 0, 0)),
        ],
        out_shape=[
            jax.ShapeDtypeStruct((B, K + 1, N), jnp.float32),
            jax.ShapeDtypeStruct((B, D, N), jnp.float32),
        ],
    )(state3, wqT, bq2, wvT, bv2)


def _sc_body(val_hbm, attn_hbm, part_hbm, out_hbm, pbuf, vbuf, abuf, obuf):
    c = lax.axis_index("c")
    s = lax.axis_index("s")
    wid = s * NC + c
    pltpu.sync_copy(part_hbm, pbuf)

    def batch_body(j, carry):
        bi = wid * PER + j
        pltpu.sync_copy(val_hbm.at[bi], vbuf)
        pltpu.sync_copy(attn_hbm.at[bi], abuf)

        def chunk_body(t, carry2):
            i0 = pl.multiple_of(t * L, L)
            a0 = abuf[0, pl.ds(i0, L)]
            a1 = abuf[1, pl.ds(i0, L)]
            a2 = abuf[2, pl.ds(i0, L)]
            p0 = pbuf[0, pl.ds(i0, L)]
            p1 = pbuf[1, pl.ds(i0, L)]
            p2 = pbuf[2, pl.ds(i0, L)]
            for d in range(D):
                dvec = jnp.full((L,), d, jnp.int32)
                g0 = plsc.load_gather(vbuf, [dvec, p0])
                g1 = plsc.load_gather(vbuf, [dvec, p1])
                g2 = plsc.load_gather(vbuf, [dvec, p2])
                obuf[d, pl.ds(i0, L)] = a0 * g0 + a1 * g1 + a2 * g2
            return carry2

        lax.fori_loop(0, NFULL, chunk_body, 0)

        posv = NFULL * L + lax.iota(jnp.int32, L)
        msk = posv < N
        posc = jnp.minimum(posv, N - 1)
        zv = jnp.zeros((L,), jnp.int32)
        a0 = plsc.load_gather(abuf, [zv, posc])
        a1 = plsc.load_gather(abuf, [zv + 1, posc])
        a2 = plsc.load_gather(abuf, [zv + 2, posc])
        p0 = plsc.load_gather(pbuf, [zv, posc])
        p1 = plsc.load_gather(pbuf, [zv + 1, posc])
        p2 = plsc.load_gather(pbuf, [zv + 2, posc])
        for d in range(D):
            dvec = jnp.full((L,), d, jnp.int32)
            g0 = plsc.load_gather(vbuf, [dvec, p0])
            g1 = plsc.load_gather(vbuf, [dvec, p1])
            g2 = plsc.load_gather(vbuf, [dvec, p2])
            plsc.store_scatter(obuf, [dvec, posc],
                               a0 * g0 + a1 * g1 + a2 * g2, mask=msk)
        pltpu.sync_copy(obuf, out_hbm.at[bi])
        return carry

    lax.fori_loop(0, PER, batch_body, 0)


def _sc_gather(values, attn, partsT):
    mesh = plsc.VectorSubcoreMesh(core_axis_name="c", subcore_axis_name="s")
    run = functools.partial(
        pl.kernel,
        mesh=mesh,
        compiler_params=pltpu.CompilerParams(
            use_tc_tiling_on_sc=False, needs_layout_passes=False),
        out_type=jax.ShapeDtypeStruct((B, D, N), jnp.float32),
        scratch_types=[
            pltpu.VMEM((K, N), jnp.int32),
            pltpu.VMEM((D, N), jnp.float32),
            pltpu.VMEM((K + 1, N), jnp.float32),
            pltpu.VMEM((D, N), jnp.float32),
        ],
    )(_sc_body)
    return run(values, attn, partsT)


def kernel(state, partners, Wq, bq, Wv, bv):
    state3 = state.reshape(B, C, N)
    wqT = Wq.T
    wvT = Wv.T
    bq2 = bq.reshape(K, 1)
    bv2 = bv.reshape(D, 1)
    partsT = partners.astype(jnp.int32).T
    attn, values = _tc_project(state3, wqT, bq2, wvT, bv2)
    out3 = _sc_gather(values, attn, partsT)
    return out3.reshape(B, D, H, W)


# speedup vs baseline: 1.0887x; 1.0887x over previous
import functools

import jax
import jax.numpy as jnp
from jax import lax
from jax.experimental import pallas as pl
from jax.experimental.pallas import tpu as pltpu
from jax.experimental.pallas import tpu_sc as plsc

H = 30
W = 30
C = 64
D = 16
K = 3
B = 1024
N = H * W

BB = 8
NC = 2
NS = 16
NW = NC * NS
PER = B // NW
L = 16
NFULL = N // L
R = D + K + 1


def _tc_proj_body(x_ref, wqT_ref, bq_ref, wvT_ref, bv_ref, comb_ref):
    wqT = wqT_ref[...]
    wvT = wvT_ref[...]
    bq = bq_ref[...]
    bv = bv_ref[...]
    for b in range(BB):
        x = x_ref[b]
        logits = jnp.dot(wqT, x, preferred_element_type=jnp.float32) + bq
        m = jnp.max(logits, axis=0, keepdims=True)
        e = jnp.exp(logits - m)
        s = jnp.sum(e, axis=0, keepdims=True)
        attn = e / s
        vals = jnp.dot(wvT, x, preferred_element_type=jnp.float32) + bv
        comb_ref[b, pl.ds(0, D), :] = vals
        comb_ref[b, pl.ds(D, K), :] = attn


def _tc_project(state3, wqT, bq2, wvT, bv2):
    return pl.pallas_call(
        _tc_proj_body,
        grid=(B // BB,),
        in_specs=[
            pl.BlockSpec((BB, C, N), lambda i: (i, 0, 0)),
            pl.BlockSpec((K, C), lambda i: (0, 0)),
            pl.BlockSpec((K, 1), lambda i: (0, 0)),
            pl.BlockSpec((D, C), lambda i: (0, 0)),
            pl.BlockSpec((D, 1), lambda i: (0, 0)),
        ],
        out_specs=pl.BlockSpec((BB, R, N), lambda i: (i, 0, 0)),
        out_shape=jax.ShapeDtypeStruct((B, R, N), jnp.float32),
        compiler_params=pltpu.CompilerParams(
            dimension_semantics=("parallel",)),
    )(state3, wqT, bq2, wvT, bv2)


def _sc_body(comb_hbm, part_hbm, out_hbm, pbuf, ibuf0, ibuf1, obuf0, obuf1,
             sin0, sin1, sout0, sout1):
    c = lax.axis_index("c")
    s = lax.axis_index("s")
    base = (s * NC + c) * PER
    pltpu.sync_copy(part_hbm, pbuf)

    ibufs = (ibuf0, ibuf1)
    obufs = (obuf0, obuf1)
    sins = (sin0, sin1)
    souts = (sout0, sout1)

    def start_in(par, j):
        pltpu.make_async_copy(comb_hbm.at[base + j], ibufs[par], sins[par]).start()

    def wait_in(par):
        pltpu.make_async_copy(comb_hbm.at[base], ibufs[par], sins[par]).wait()

    def start_out(par, j):
        pltpu.make_async_copy(obufs[par], out_hbm.at[base + j], souts[par]).start()

    def wait_out(par):
        pltpu.make_async_copy(obufs[par], out_hbm.at[base], souts[par]).wait()

    def compute(ibuf, obuf):
        def chunk_body(t, carry):
            i0 = pl.multiple_of(t * L, L)
            a0 = ibuf[D + 0, pl.ds(i0, L)]
            a1 = ibuf[D + 1, pl.ds(i0, L)]
            a2 = ibuf[D + 2, pl.ds(i0, L)]
            p0 = pbuf[0, pl.ds(i0, L)]
            p1 = pbuf[1, pl.ds(i0, L)]
            p2 = pbuf[2, pl.ds(i0, L)]
            for d in range(D):
                dvec = jnp.full((L,), d, jnp.int32)
                g0 = plsc.load_gather(ibuf, [dvec, p0])
                g1 = plsc.load_gather(ibuf, [dvec, p1])
                g2 = plsc.load_gather(ibuf, [dvec, p2])
                obuf[d, pl.ds(i0, L)] = a0 * g0 + a1 * g1 + a2 * g2
            return carry

        lax.fori_loop(0, NFULL, chunk_body, 0)

        posv = NFULL * L + lax.iota(jnp.int32, L)
        msk = posv < N
        posc = jnp.minimum(posv, N - 1)
        zv = jnp.zeros((L,), jnp.int32)
        a0 = plsc.load_gather(ibuf, [zv + D, posc])
        a1 = plsc.load_gather(ibuf, [zv + D + 1, posc])
        a2 = plsc.load_gather(ibuf, [zv + D + 2, posc])
        p0 = plsc.load_gather(pbuf, [zv, posc])
        p1 = plsc.load_gather(pbuf, [zv + 1, posc])
        p2 = plsc.load_gather(pbuf, [zv + 2, posc])
        for d in range(D):
            dvec = jnp.full((L,), d, jnp.int32)
            g0 = plsc.load_gather(ibuf, [dvec, p0])
            g1 = plsc.load_gather(ibuf, [dvec, p1])
            g2 = plsc.load_gather(ibuf, [dvec, p2])
            plsc.store_scatter(obuf, [dvec, posc],
                               a0 * g0 + a1 * g1 + a2 * g2, mask=msk)

    start_in(0, 0)
    start_in(1, 1)

    def outer(t, carry):
        j0 = t * 2
        for par in range(2):
            j = j0 + par
            wait_in(par)

            @pl.when(j >= 2)
            def _():
                wait_out(par)

            compute(ibufs[par], obufs[par])
            start_out(par, j)

            @pl.when(j + 2 < PER)
            def _():
                start_in(par, j + 2)
        return carry

    lax.fori_loop(0, PER // 2, outer, 0)
    wait_out(0)
    wait_out(1)


def _sc_gather(comb, partsT):
    mesh = plsc.VectorSubcoreMesh(core_axis_name="c", subcore_axis_name="s")
    run = functools.partial(
        pl.kernel,
        mesh=mesh,
        compiler_params=pltpu.CompilerParams(
            use_tc_tiling_on_sc=False, needs_layout_passes=False),
        out_type=jax.ShapeDtypeStruct((B, D, N), jnp.float32),
        scratch_types=[
            pltpu.VMEM((K, N), jnp.int32),
            pltpu.VMEM((R, N), jnp.float32),
            pltpu.VMEM((R, N), jnp.float32),
            pltpu.VMEM((D, N), jnp.float32),
            pltpu.VMEM((D, N), jnp.float32),
            pltpu.SemaphoreType.DMA,
            pltpu.SemaphoreType.DMA,
            pltpu.SemaphoreType.DMA,
            pltpu.SemaphoreType.DMA,
        ],
    )(_sc_body)
    return run(comb, partsT)


def kernel(state, partners, Wq, bq, Wv, bv):
    state3 = state.reshape(B, C, N)
    wqT = Wq.T
    wvT = Wv.T
    bq2 = bq.reshape(K, 1)
    bv2 = bv.reshape(D, 1)
    partsT = partners.astype(jnp.int32).T
    comb = _tc_project(state3, wqT, bq2, wvT, bv2)
    out3 = _sc_gather(comb, partsT)
    return out3.reshape(B, D, H, W)


# speedup vs baseline: 1.3417x; 1.2324x over previous
import functools

import jax
import jax.numpy as jnp
from jax import lax
from jax.experimental import pallas as pl
from jax.experimental.pallas import tpu as pltpu
from jax.experimental.pallas import tpu_sc as plsc

H = 30
W = 30
C = 64
D = 16
K = 3
B = 1024
N = H * W

BB = 8
G = B // BB
NC = 2
NS = 16
NW = NC * NS
PER = B // NW
L = 16
NP = 1024
NFULL = N // L
R = D + K + 1


def _tc_proj_body(x_hbm, wqT_ref, bq_ref, wvT_ref, bv_ref, comb_ref,
                  xbuf0, xbuf1, sem0, sem1):
    s = pl.program_id(0)
    xbufs = (xbuf0, xbuf1)
    sems = (sem0, sem1)

    def copy_step(step, par):
        src = x_hbm.at[pl.ds(step * BB, BB)]
        return pltpu.make_async_copy(src, xbufs[par], sems[par])

    @pl.when(s == 0)
    def _():
        copy_step(0, 0).start()

    wqT = wqT_ref[...]
    wvT = wvT_ref[...]
    bq = bq_ref[...]
    bv = bv_ref[...]

    def do_par(par):
        copy_step(s, par).wait()

        @pl.when(s + 1 < G)
        def _():
            copy_step(s + 1, 1 - par).start()

        xbuf = xbufs[par]
        for b in range(BB):
            x = xbuf[b]
            logits = jnp.dot(wqT, x, preferred_element_type=jnp.float32) + bq
            m = jnp.max(logits, axis=0, keepdims=True)
            e = jnp.exp(logits - m)
            ssum = jnp.sum(e, axis=0, keepdims=True)
            attn = e / ssum
            vals = jnp.dot(wvT, x, preferred_element_type=jnp.float32) + bv
            comb_ref[pl.ds(b * R, D), pl.ds(0, N)] = vals
            comb_ref[pl.ds(b * R + D, K), pl.ds(0, N)] = attn

    @pl.when(s % 2 == 0)
    def _():
        do_par(0)

    @pl.when(s % 2 == 1)
    def _():
        do_par(1)


def _tc_project(state2, wqT, bq2, wvT, bv2):
    return pl.pallas_call(
        _tc_proj_body,
        grid=(G,),
        in_specs=[
            pl.BlockSpec(memory_space=pl.ANY),
            pl.BlockSpec((K, C), lambda i: (0, 0)),
            pl.BlockSpec((K, 1), lambda i: (0, 0)),
            pl.BlockSpec((D, C), lambda i: (0, 0)),
            pl.BlockSpec((D, 1), lambda i: (0, 0)),
        ],
        out_specs=pl.BlockSpec((BB * R, NP), lambda i: (i, 0)),
        out_shape=jax.ShapeDtypeStruct((B * R, NP), jnp.float32),
        scratch_shapes=[
            pltpu.VMEM((BB, C, N), jnp.float32),
            pltpu.VMEM((BB, C, N), jnp.float32),
            pltpu.SemaphoreType.DMA,
            pltpu.SemaphoreType.DMA,
        ],
        compiler_params=pltpu.CompilerParams(
            dimension_semantics=("arbitrary",)),
    )(state2, wqT, bq2, wvT, bv2)


def _sc_body(comb_hbm, part_hbm, out_hbm, pbuf, ibuf0, ibuf1, obuf0, obuf1,
             sin0, sin1, sout0, sout1):
    c = lax.axis_index("c")
    s = lax.axis_index("s")
    base = (s * NC + c) * PER
    pltpu.sync_copy(part_hbm, pbuf)

    ibufs = (ibuf0, ibuf1)
    obufs = (obuf0, obuf1)
    sins = (sin0, sin1)
    souts = (sout0, sout1)

    def start_in(par, j):
        pltpu.make_async_copy(comb_hbm.at[base + j], ibufs[par], sins[par]).start()

    def wait_in(par):
        pltpu.make_async_copy(comb_hbm.at[base], ibufs[par], sins[par]).wait()

    def start_out(par, j):
        pltpu.make_async_copy(obufs[par], out_hbm.at[base + j], souts[par]).start()

    def wait_out(par):
        pltpu.make_async_copy(obufs[par], out_hbm.at[base], souts[par]).wait()

    def compute(ibuf, obuf):
        @plsc.parallel_loop(0, NFULL * L, L, unroll=1)
        def chunk_body(i0):
            a0 = ibuf[pl.ds(pl.multiple_of(D * NP + i0, L), L)]
            a1 = ibuf[pl.ds(pl.multiple_of((D + 1) * NP + i0, L), L)]
            a2 = ibuf[pl.ds(pl.multiple_of((D + 2) * NP + i0, L), L)]
            p0 = pbuf[pl.ds(pl.multiple_of(i0, L), L)]
            p1 = pbuf[pl.ds(pl.multiple_of(NP + i0, L), L)]
            p2 = pbuf[pl.ds(pl.multiple_of(2 * NP + i0, L), L)]
            for d in range(D):
                off = d * NP
                g0 = plsc.load_gather(ibuf, [p0 + off])
                g1 = plsc.load_gather(ibuf, [p1 + off])
                g2 = plsc.load_gather(ibuf, [p2 + off])
                obuf[d, pl.ds(pl.multiple_of(i0, L), L)] = (
                    a0 * g0 + a1 * g1 + a2 * g2)

        t0 = NFULL * L
        posv = t0 + lax.iota(jnp.int32, L)
        msk = posv < N
        posc = jnp.minimum(posv, N - 1)
        a0 = ibuf[pl.ds(D * NP + t0, L)]
        a1 = ibuf[pl.ds((D + 1) * NP + t0, L)]
        a2 = ibuf[pl.ds((D + 2) * NP + t0, L)]
        p0 = pbuf[pl.ds(t0, L)]
        p1 = pbuf[pl.ds(NP + t0, L)]
        p2 = pbuf[pl.ds(2 * NP + t0, L)]
        for d in range(D):
            off = d * NP
            dvec = jnp.full((L,), d, jnp.int32)
            g0 = plsc.load_gather(ibuf, [p0 + off])
            g1 = plsc.load_gather(ibuf, [p1 + off])
            g2 = plsc.load_gather(ibuf, [p2 + off])
            plsc.store_scatter(obuf, [dvec, posc],
                               a0 * g0 + a1 * g1 + a2 * g2, mask=msk)

    start_in(0, 0)
    start_in(1, 1)

    def outer(t, carry):
        j0 = t * 2
        for par in range(2):
            j = j0 + par
            wait_in(par)

            @pl.when(j >= 2)
            def _():
                wait_out(par)

            compute(ibufs[par], obufs[par])
            start_out(par, j)

            @pl.when(j + 2 < PER)
            def _():
                start_in(par, j + 2)
        return carry

    lax.fori_loop(0, PER // 2, outer, 0)
    wait_out(0)
    wait_out(1)


def _sc_gather(comb2, partsF):
    mesh = plsc.VectorSubcoreMesh(core_axis_name="c", subcore_axis_name="s")
    run = functools.partial(
        pl.kernel,
        mesh=mesh,
        compiler_params=pltpu.CompilerParams(
            use_tc_tiling_on_sc=False, needs_layout_passes=False),
        out_type=jax.ShapeDtypeStruct((B, D, N), jnp.float32),
        scratch_types=[
            pltpu.VMEM((K * NP,), jnp.int32),
            pltpu.VMEM((R * NP,), jnp.float32),
            pltpu.VMEM((R * NP,), jnp.float32),
            pltpu.VMEM((D, N), jnp.float32),
            pltpu.VMEM((D, N), jnp.float32),
            pltpu.SemaphoreType.DMA,
            pltpu.SemaphoreType.DMA,
            pltpu.SemaphoreType.DMA,
            pltpu.SemaphoreType.DMA,
        ],
    )(_sc_body)
    return run(comb2, partsF)


def kernel(state, partners, Wq, bq, Wv, bv):
    state2 = state.reshape(B, C, N)
    wqT = Wq.T
    wvT = Wv.T
    bq2 = bq.reshape(K, 1)
    bv2 = bv.reshape(D, 1)
    partsF = (jnp.zeros((K, NP), jnp.int32)
              .at[:, :N].set(partners.astype(jnp.int32).T)
              .reshape(K * NP))
    comb = _tc_project(state2, wqT, bq2, wvT, bv2)
    out3 = _sc_gather(comb.reshape(B, R * NP), partsF)
    return out3.reshape(B, D, H, W)
